# baseline (speedup 1.0000x reference)
import functools

import jax
import jax.numpy as jnp
from jax import lax
from jax.experimental import pallas as pl
from jax.experimental.pallas import tpu as pltpu
from jax.experimental.pallas import tpu_sc as plsc

_NC = 2
_NS = 16
_NW = _NC * _NS
_LANES = 16
_CHUNK = 100
_NBUF = 4
_PRE = 2


@functools.lru_cache(maxsize=None)
def _build(batch, seqlen, vocab, embed):
    rows = batch * seqlen
    assert rows % (_NW * _CHUNK) == 0
    rpw = rows // _NW
    nsub = rpw // _CHUNK
    assert nsub % _NBUF == 0
    assert rpw % seqlen == 0
    assert seqlen % _CHUNK == 0
    assert embed % _LANES == 0
    nq = embed // _LANES
    noff = seqlen // _CHUNK

    mesh = plsc.VectorSubcoreMesh(core_axis_name="c", subcore_axis_name="s")

    @functools.partial(
        pl.kernel,
        out_type=jax.ShapeDtypeStruct((rows // _CHUNK, _CHUNK, embed), jnp.float32),
        mesh=mesh,
        compiler_params=pltpu.CompilerParams(use_tc_tiling_on_sc=False),
        scratch_types=[
            pltpu.VMEM((nsub, _CHUNK), jnp.int32),
            pltpu.VMEM((_NBUF, _CHUNK, embed), jnp.float32),
        ] + [pltpu.SemaphoreType.DMA] * (3 * _NBUF),
    )
    def fused(x_hbm, tok_hbm, pos_hbm, out_hbm, idx_v, rows_v, *sems):
        psem = sems[0:_NBUF]
        gsem = sems[_NBUF:2 * _NBUF]
        osem = sems[2 * _NBUF:3 * _NBUF]
        cid = lax.axis_index("c")
        sid = lax.axis_index("s")
        wid = sid * _NC + cid
        pltpu.sync_copy(x_hbm.at[wid], idx_v)

        def pstart(t, b):
            off = lax.rem(t, noff) * _CHUNK
            pltpu.async_copy(
                pos_hbm.at[pl.ds(off, _CHUNK)], rows_v.at[b], psem[b]
            )

        def pwait(b):
            pltpu.make_async_copy(
                pos_hbm.at[pl.ds(0, _CHUNK)], rows_v.at[b], psem[b]
            ).wait()

        def gstart(t, b):
            pltpu.async_copy(
                tok_hbm.at[idx_v.at[t]], rows_v.at[b], gsem[b], add=True
            )

        def gwait(t, b):
            pltpu.make_async_copy(
                tok_hbm.at[idx_v.at[t]], rows_v.at[b], gsem[b]
            ).wait()

        def ostart(t, b):
            pltpu.async_copy(rows_v.at[b], out_hbm.at[wid * nsub + t], osem[b])

        def owait(b):
            pltpu.make_async_copy(rows_v.at[b], out_hbm.at[0], osem[b]).wait()

        pstart(0, 0)
        pstart(1, 1)
        pwait(0)
        gstart(0, 0)

        def outer(i, carry):
            t0 = i * _NBUF
            for b in range(_NBUF):
                t = t0 + b
                gwait(t, b)
                ostart(t, b)

                v = t + 2
                bv = (b + 2) % _NBUF

                @pl.when(v < nsub)
                def _():
                    @pl.when(v >= _NBUF)
                    def _():
                        owait(bv)

                    pstart(v, bv)

                w = t + 1
                bw = (b + 1) % _NBUF

                @pl.when(w < nsub)
                def _():
                    pwait(bw)
                    gstart(w, bw)

            return carry

        lax.fori_loop(0, nsub // _NBUF, outer, None)
        for b in range(_NBUF):
            owait(b)

    return fused


def kernel(x, token_table, pos_table):
    batch, seqlen = x.shape
    vocab, embed = token_table.shape
    fused = _build(batch, seqlen, vocab, embed)
    rows = batch * seqlen
    x3 = x.astype(jnp.int32).reshape(_NW, rows // (_NW * _CHUNK), _CHUNK)
    out = fused(x3, token_table, pos_table)
    return out.reshape(batch, seqlen, embed)


# speedup vs baseline: 1.4911x; 1.4911x over previous
import functools

import jax
import jax.numpy as jnp
from jax import lax
from jax.experimental import pallas as pl
from jax.experimental.pallas import tpu as pltpu
from jax.experimental.pallas import tpu_sc as plsc

_NC = 2
_NS = 16
_NW = _NC * _NS
_LANES = 16
_CHUNK = 100
_NBUF = 8
_PRE = 4


@functools.lru_cache(maxsize=None)
def _build(batch, seqlen, vocab, embed):
    rows = batch * seqlen
    assert rows % (_NW * _CHUNK) == 0
    rpw = rows // _NW
    nsub = rpw // _CHUNK
    assert nsub % _NBUF == 0
    assert rpw % seqlen == 0
    assert seqlen % _CHUNK == 0
    assert embed % _LANES == 0
    nq = embed // _LANES
    noff = seqlen // _CHUNK

    mesh = plsc.VectorSubcoreMesh(core_axis_name="c", subcore_axis_name="s")

    @functools.partial(
        pl.kernel,
        out_type=jax.ShapeDtypeStruct((rows // _CHUNK, _CHUNK, embed), jnp.float32),
        mesh=mesh,
        compiler_params=pltpu.CompilerParams(use_tc_tiling_on_sc=False),
        scratch_types=[
            pltpu.VMEM((nsub, _CHUNK), jnp.int32),
            pltpu.VMEM((seqlen, embed), jnp.float32),
            pltpu.VMEM((_NBUF, _CHUNK, embed), jnp.float32),
        ] + [pltpu.SemaphoreType.DMA] * (2 * _NBUF),
    )
    def fused(x_hbm, tok_hbm, pos_hbm, out_hbm, idx_v, pos_v, rows_v, *sems):
        gsem = sems[0:_NBUF]
        osem = sems[_NBUF:2 * _NBUF]
        cid = lax.axis_index("c")
        sid = lax.axis_index("s")
        wid = sid * _NC + cid
        pltpu.sync_copy(x_hbm.at[wid], idx_v)
        pltpu.sync_copy(pos_hbm, pos_v)

        def gstart(t, b):
            pltpu.async_copy(tok_hbm.at[idx_v.at[t]], rows_v.at[b], gsem[b])

        def gwait(t, b):
            pltpu.make_async_copy(
                tok_hbm.at[idx_v.at[t]], rows_v.at[b], gsem[b]
            ).wait()

        def ostart(t, b):
            pltpu.async_copy(rows_v.at[b], out_hbm.at[wid * nsub + t], osem[b])

        def owait(b):
            pltpu.make_async_copy(rows_v.at[b], out_hbm.at[0], osem[b]).wait()

        for b in range(_PRE):
            gstart(b, b)

        def outer(i, carry):
            t0 = i * _NBUF
            for b in range(_NBUF):
                t = t0 + b
                gwait(t, b)
                off = lax.rem(t, noff) * _CHUNK

                def addrow(j, c2):
                    for q in range(nq):
                        sl = pl.ds(q * _LANES, _LANES)
                        rows_v[b, j, sl] = rows_v[b, j, sl] + pos_v[off + j, sl]
                    return c2

                lax.fori_loop(0, _CHUNK, addrow, None)

                ostart(t, b)
                u = t + _PRE
                bu = (b + _PRE) % _NBUF

                @pl.when(u < nsub)
                def _():
                    @pl.when(u >= _NBUF)
                    def _():
                        owait(bu)

                    gstart(u, bu)

            return carry

        lax.fori_loop(0, nsub // _NBUF, outer, None)
        for b in range(_NBUF):
            owait(b)

    return fused


def kernel(x, token_table, pos_table):
    batch, seqlen = x.shape
    vocab, embed = token_table.shape
    fused = _build(batch, seqlen, vocab, embed)
    rows = batch * seqlen
    x3 = x.astype(jnp.int32).reshape(_NW, rows // (_NW * _CHUNK), _CHUNK)
    out = fused(x3, token_table, pos_table)
    return out.reshape(batch, seqlen, embed)


# speedup vs baseline: 1.6226x; 1.0882x over previous
import functools

import jax
import jax.numpy as jnp
from jax import lax
from jax.experimental import pallas as pl
from jax.experimental.pallas import tpu as pltpu
from jax.experimental.pallas import tpu_sc as plsc

_NC = 2
_NS = 16
_NW = _NC * _NS
_LANES = 16
_PAD = 128
_CHUNK = 100
_NBUF = 2
_PRE = 2
_STEP = 4
_NSTG = 2


@functools.lru_cache(maxsize=None)
def _build(batch, seqlen, vocab, embed):
    rows = batch * seqlen
    bpw = batch // _NW
    hpw = rows // (_NW * _CHUNK)
    assert batch % _NW == 0
    assert seqlen == 2 * _CHUNK
    assert hpw % _STEP == 0
    assert embed % _LANES == 0
    nq = embed // _LANES

    mesh = plsc.VectorSubcoreMesh(core_axis_name="c", subcore_axis_name="s")

    @functools.partial(
        pl.kernel,
        out_type=jax.ShapeDtypeStruct((batch, seqlen, embed), jnp.float32),
        mesh=mesh,
        scratch_types=[
            pltpu.VMEM((hpw, _CHUNK), jnp.int32),
            pltpu.VMEM((seqlen, embed), jnp.float32),
            pltpu.VMEM((_NBUF, _CHUNK, _PAD), jnp.float32),
            pltpu.VMEM((_NSTG, seqlen, embed), jnp.float32),
        ] + [pltpu.SemaphoreType.DMA] * (_NBUF + _NSTG),
    )
    def fused(x_hbm, tok_hbm, pos_hbm, out_hbm, idx_v, pos_v, rows_v, stg_v,
              *sems):
        gsem = sems[:_NBUF]
        osem = sems[_NBUF:]
        cid = lax.axis_index("c")
        sid = lax.axis_index("s")
        wid = sid * _NC + cid
        pltpu.sync_copy(x_hbm.at[wid], idx_v)
        pltpu.sync_copy(pos_hbm, pos_v)

        def gstart(h, b):
            pltpu.async_copy(tok_hbm.at[idx_v.at[h]], rows_v.at[b], gsem[b])

        def gwait(h, b):
            pltpu.make_async_copy(
                tok_hbm.at[idx_v.at[h]], rows_v.at[b], gsem[b]
            ).wait()

        def ostart(c, o):
            pltpu.async_copy(stg_v.at[o], out_hbm.at[wid * bpw + c], osem[o])

        def owait(o):
            pltpu.make_async_copy(stg_v.at[o], out_hbm.at[0], osem[o]).wait()

        for b in range(_PRE):
            gstart(b, b)

        def outer(i, carry):
            h0 = i * _STEP
            for k in range(_STEP):
                h = h0 + k
                half = k % 2
                o = k // 2
                gwait(h, k % _NBUF)

                if half == 0:
                    @pl.when(i >= 1)
                    def _():
                        owait(o)

                soff = half * _CHUNK

                def addrow(j, c2):
                    for q in range(nq):
                        sl = pl.ds(q * _LANES, _LANES)
                        stg_v[o, soff + j, sl] = (
                            rows_v[k % _NBUF, j, sl] + pos_v[soff + j, sl]
                        )
                    return c2

                lax.fori_loop(0, _CHUNK, addrow, None)

                if half == 1:
                    ostart(i * 2 + o, o)

                u = h + _PRE
                bu = (k + _PRE) % _NBUF

                @pl.when(u < hpw)
                def _():
                    gstart(u, bu)

            return carry

        lax.fori_loop(0, hpw // _STEP, outer, None)
        for o in range(_NSTG):
            owait(o)

    return fused


def kernel(x, token_table, pos_table):
    batch, seqlen = x.shape
    vocab, embed = token_table.shape
    fused = _build(batch, seqlen, vocab, embed)
    rows = batch * seqlen
    x3 = x.astype(jnp.int32).reshape(_NW, rows // (_NW * _CHUNK), _CHUNK)
    tok_pad = jnp.pad(token_table, ((0, 0), (0, _PAD - embed)))
    return fused(x3, tok_pad, pos_table)
